# baseline (speedup 1.0000x reference)
import jax
import jax.numpy as jnp
from jax.experimental import pallas as pl
from jax.experimental.pallas import tpu as pltpu

_BLOCK = 1024
_NSTREAM = 4
_NDEPTH = 2
_NE = 8


def _top2_block(x, wt):
    logits = jnp.dot(x, wt, preferred_element_type=jnp.float32)
    lane = jax.lax.broadcasted_iota(jnp.int32, logits.shape, 1)
    l1 = jnp.max(logits, axis=-1, keepdims=True)
    i1 = jnp.argmax(logits, axis=-1).astype(jnp.int32)[:, None]
    masked = jnp.where(lane == i1, -jnp.inf, logits)
    l2 = jnp.max(masked, axis=-1, keepdims=True)
    i2 = jnp.argmax(masked, axis=-1).astype(jnp.int32)[:, None]
    t = jnp.exp(l2 - l1)
    w1 = 1.0 / (1.0 + t)
    w2 = t * w1
    idx = jnp.concatenate([i1, i2], axis=1)
    w = jnp.concatenate([w1, w2], axis=1)
    return idx, w


def _gate_body(x_hbm, wt_ref, idx_ref, w_ref, *scratch):
    bufs = scratch[:_NSTREAM]
    sems = scratch[_NSTREAM:]
    n = x_hbm.shape[0]
    nch = n // _BLOCK
    nbuf = _NSTREAM * _NDEPTH

    def copy(j, st, d):
        return pltpu.make_async_copy(
            x_hbm.at[pl.ds(j * _BLOCK, _BLOCK), :],
            bufs[st].at[d],
            sems[st].at[d],
        )

    for j in range(nbuf):
        copy(j, j % _NSTREAM, (j // _NSTREAM) % _NDEPTH).start()

    def loop(jo, carry):
        for s in range(nbuf):
            j = jo * nbuf + s
            st = s % _NSTREAM
            d = (s // _NSTREAM) % _NDEPTH
            copy(j, st, d).wait()
            idx, w = _top2_block(bufs[st][d], wt_ref[...])
            idx_ref[pl.ds(j * _BLOCK, _BLOCK), :] = idx
            w_ref[pl.ds(j * _BLOCK, _BLOCK), :] = w

            @pl.when(j + nbuf < nch)
            def _():
                copy(j + nbuf, st, d).start()

        return carry

    jax.lax.fori_loop(0, nch // nbuf, loop, 0)


def _route(x, wt):
    n, h = x.shape
    return pl.pallas_call(
        _gate_body,
        in_specs=[
            pl.BlockSpec(memory_space=pl.ANY),
            pl.BlockSpec(memory_space=pltpu.VMEM),
        ],
        out_specs=[
            pl.BlockSpec(memory_space=pltpu.VMEM),
            pl.BlockSpec(memory_space=pltpu.VMEM),
        ],
        out_shape=[
            jax.ShapeDtypeStruct((n, 2), jnp.int32),
            jax.ShapeDtypeStruct((n, 2), jnp.float32),
        ],
        scratch_shapes=(
            [pltpu.VMEM((_NDEPTH, _BLOCK, h), jnp.float32) for _ in range(_NSTREAM)]
            + [pltpu.SemaphoreType.DMA((_NDEPTH,)) for _ in range(_NSTREAM)]
        ),
    )(x, wt)


@jax.jit
def kernel(hidden_states, weight):
    h = hidden_states.shape[-1]
    x = hidden_states.reshape(-1, h)
    topk_idx, topk_weight = _route(x, weight.T)
    return topk_idx, topk_weight


# speedup vs baseline: 1.1137x; 1.1137x over previous
import jax
import jax.numpy as jnp
from jax.experimental import pallas as pl
from jax.experimental.pallas import tpu as pltpu

_BLOCK = 2048
_NSPLIT = 3
_NE = 8


def _gate_body(*refs):
    xs = refs[:_NSPLIT]
    wt_ref = refs[_NSPLIT]
    idx_ref, w_ref = refs[_NSPLIT + 1], refs[_NSPLIT + 2]
    hs = xs[0].shape[1]
    logits = jnp.dot(xs[0][...], wt_ref[0 : hs, :], preferred_element_type=jnp.float32)
    for k in range(1, _NSPLIT):
        logits = logits + jnp.dot(
            xs[k][...], wt_ref[k * hs : (k + 1) * hs, :],
            preferred_element_type=jnp.float32,
        )
    lane = jax.lax.broadcasted_iota(jnp.int32, logits.shape, 1)
    l1 = jnp.max(logits, axis=-1, keepdims=True)
    i1 = jnp.argmax(logits, axis=-1).astype(jnp.int32)[:, None]
    masked = jnp.where(lane == i1, -jnp.inf, logits)
    l2 = jnp.max(masked, axis=-1, keepdims=True)
    i2 = jnp.argmax(masked, axis=-1).astype(jnp.int32)[:, None]
    t = jnp.exp(l2 - l1)
    w1 = 1.0 / (1.0 + t)
    w2 = t * w1
    idx_ref[...] = jnp.concatenate([i1, i2], axis=1)
    w_ref[...] = jnp.concatenate([w1, w2], axis=1)


def _route(x, wt):
    n, h = x.shape
    hs = h // _NSPLIT
    grid = n // _BLOCK
    return pl.pallas_call(
        _gate_body,
        grid=(grid,),
        in_specs=(
            [
                pl.BlockSpec((_BLOCK, hs), lambda i, k=k: (i, k))
                for k in range(_NSPLIT)
            ]
            + [pl.BlockSpec((h, _NE), lambda i: (0, 0))]
        ),
        out_specs=[
            pl.BlockSpec((_BLOCK, 2), lambda i: (i, 0)),
            pl.BlockSpec((_BLOCK, 2), lambda i: (i, 0)),
        ],
        out_shape=[
            jax.ShapeDtypeStruct((n, 2), jnp.int32),
            jax.ShapeDtypeStruct((n, 2), jnp.float32),
        ],
        compiler_params=pltpu.CompilerParams(
            dimension_semantics=("arbitrary",),
        ),
    )(*([x] * _NSPLIT), wt)


@jax.jit
def kernel(hidden_states, weight):
    h = hidden_states.shape[-1]
    x = hidden_states.reshape(-1, h)
    topk_idx, topk_weight = _route(x, weight.T)
    return topk_idx, topk_weight
